# Initial kernel scaffold; baseline (speedup 1.0000x reference)
import functools

import jax
import jax.numpy as jnp
from jax import lax
from jax.experimental import pallas as pl
from jax.experimental.pallas import tpu as pltpu
from jax.experimental.pallas import tpu_sc as plsc

_NC = 2
_NS = 16
_LANES = 16


def _mesh():
    return plsc.VectorSubcoreMesh(
        core_axis_name="c", subcore_axis_name="s", num_cores=_NC, num_subcores=_NS
    )


def _make_deg_kernel(rows, npad):
    groups = rows // 8
    seg = npad // _NS

    @functools.partial(
        pl.kernel,
        mesh=_mesh(),
        out_type=jax.ShapeDtypeStruct((_NC, npad), jnp.float32),
        scratch_types=[
            pltpu.VMEM((8, 128), jnp.int32),
            pltpu.VMEM((128,), jnp.float32),
            pltpu.VMEM_SHARED((npad,), jnp.float32),
        ],
    )
    def deg_kernel(e_hbm, zero_hbm, out_hbm, dstv, onesv, accum):
        cid = lax.axis_index("c")
        sid = lax.axis_index("s")
        wid = sid * _NC + cid
        pltpu.sync_copy(zero_hbm.at[pl.ds(sid * seg, seg)],
                        accum.at[pl.ds(sid * seg, seg)])
        one16 = jnp.full((_LANES,), 1.0, jnp.float32)
        for k in range(8):
            onesv[pl.ds(k * _LANES, _LANES)] = one16
        plsc.subcore_barrier()
        g0 = (wid * groups) // (_NC * _NS)
        g1 = ((wid + 1) * groups) // (_NC * _NS)

        @pl.loop(g0, g1)
        def _(g):
            pltpu.sync_copy(e_hbm.at[1, pl.ds(g * 8, 8)], dstv)
            for r in range(8):
                pltpu.sync_copy(onesv, accum.at[dstv.at[r]], add=True)

        plsc.subcore_barrier()
        pltpu.sync_copy(accum.at[pl.ds(sid * seg, seg)],
                        out_hbm.at[cid, pl.ds(sid * seg, seg)])

    return deg_kernel


def _make_agg_kernel(rows, npad):
    groups = rows // 8
    seg = npad // _NS

    @functools.partial(
        pl.kernel,
        mesh=_mesh(),
        out_type=jax.ShapeDtypeStruct((_NC, npad), jnp.float32),
        scratch_types=[
            pltpu.VMEM((8, 128), jnp.int32),
            pltpu.VMEM((8, 128), jnp.int32),
            pltpu.VMEM((128,), jnp.float32),
            pltpu.VMEM((npad,), jnp.float32),
            pltpu.VMEM_SHARED((npad,), jnp.float32),
        ],
    )
    def agg_kernel(e_hbm, tab_hbm, zero_hbm, out_hbm, srcv, dstv, valsv, table,
                   accum):
        cid = lax.axis_index("c")
        sid = lax.axis_index("s")
        wid = sid * _NC + cid
        pltpu.sync_copy(zero_hbm.at[pl.ds(sid * seg, seg)],
                        accum.at[pl.ds(sid * seg, seg)])
        pltpu.sync_copy(tab_hbm, table)
        plsc.subcore_barrier()
        g0 = (wid * groups) // (_NC * _NS)
        g1 = ((wid + 1) * groups) // (_NC * _NS)

        @pl.loop(g0, g1)
        def _(g):
            pltpu.sync_copy(e_hbm.at[0, pl.ds(g * 8, 8)], srcv)
            pltpu.sync_copy(e_hbm.at[1, pl.ds(g * 8, 8)], dstv)
            for r in range(8):
                for c in range(8):
                    idx = srcv[r, pl.ds(c * _LANES, _LANES)]
                    vals = plsc.load_gather(table, [idx])
                    valsv[pl.ds(c * _LANES, _LANES)] = vals
                pltpu.sync_copy(valsv, accum.at[dstv.at[r]], add=True)

        plsc.subcore_barrier()
        pltpu.sync_copy(accum.at[pl.ds(sid * seg, seg)],
                        out_hbm.at[cid, pl.ds(sid * seg, seg)])

    return agg_kernel


def _tc_prep(deg_parts, xpad):
    r128 = xpad.shape

    def body(dref, xref, dis_ref, xd_ref):
        deg = dref[0] + dref[1]
        dis = jnp.where(deg > 0, lax.rsqrt(jnp.maximum(deg, 1e-12)),
                        jnp.zeros_like(deg))
        dis_ref[...] = dis
        xd_ref[...] = dis * xref[...]

    return pl.pallas_call(
        body,
        out_shape=(
            jax.ShapeDtypeStruct(r128, jnp.float32),
            jax.ShapeDtypeStruct(r128, jnp.float32),
        ),
    )(deg_parts, xpad)


def _tc_mid(t1_parts, dis, W1, b1, W2):
    r128 = dis.shape

    def body(tref, dis_ref, w1_ref, b1_ref, w2_ref, hd_ref):
        d = dis_ref[...]
        s1 = d * (tref[0] + tref[1])
        acc = jnp.zeros_like(s1)
        for j in range(16):
            acc = acc + jnp.maximum(s1 * w1_ref[0, j] + b1_ref[j], 0.0) * w2_ref[j, 0]
        hd_ref[...] = acc * d

    return pl.pallas_call(
        body,
        in_specs=[
            pl.BlockSpec(),
            pl.BlockSpec(),
            pl.BlockSpec(memory_space=pltpu.SMEM),
            pl.BlockSpec(memory_space=pltpu.SMEM),
            pl.BlockSpec(memory_space=pltpu.SMEM),
        ],
        out_shape=jax.ShapeDtypeStruct(r128, jnp.float32),
    )(t1_parts, dis, W1, b1, W2)


def _tc_final(t2_parts, dis, b2):
    r128 = dis.shape

    def body(tref, dis_ref, b2_ref, out_ref):
        out_ref[...] = dis_ref[...] * (tref[0] + tref[1]) + b2_ref[0]

    return pl.pallas_call(
        body,
        in_specs=[
            pl.BlockSpec(),
            pl.BlockSpec(),
            pl.BlockSpec(memory_space=pltpu.SMEM),
        ],
        out_shape=jax.ShapeDtypeStruct(r128, jnp.float32),
    )(t2_parts, dis, b2)


def kernel(x, edge_index, W1, b1, W2, b2):
    n = x.shape[0]
    e = edge_index.shape[1]
    assert e % 1024 == 0
    rows = e // 128
    npad = ((n + 1023) // 1024) * 1024
    r = npad // 128

    ei = edge_index.astype(jnp.int32).reshape(2, rows, 128)
    zeros_np = jnp.zeros((npad,), jnp.float32)
    xpad = jnp.concatenate([x[:, 0], jnp.zeros((npad - n,), jnp.float32)])

    deg_parts = _make_deg_kernel(rows, npad)(ei, zeros_np)
    dis, xd = _tc_prep(deg_parts.reshape(2, r, 128), xpad.reshape(r, 128))

    agg = _make_agg_kernel(rows, npad)
    t1_parts = agg(ei, xd.reshape(npad), zeros_np)
    hd = _tc_mid(t1_parts.reshape(2, r, 128), dis, W1, b1, W2)

    t2_parts = agg(ei, hd.reshape(npad), zeros_np)
    out = _tc_final(t2_parts.reshape(2, r, 128), dis, b2)

    return out.reshape(npad)[:n].reshape(n, 1)


# speedup vs baseline: 186.2076x; 186.2076x over previous
import functools

import jax
import jax.numpy as jnp
from jax import lax
from jax.experimental import pallas as pl
from jax.experimental.pallas import tpu as pltpu
from jax.experimental.pallas import tpu_sc as plsc

_NC = 2
_NS = 16
_LANES = 16


def _mesh():
    return plsc.VectorSubcoreMesh(
        core_axis_name="c", subcore_axis_name="s", num_cores=_NC, num_subcores=_NS
    )


def _make_deg_kernel(rows, npad):
    groups = rows // 8
    seg = npad // _NS

    @functools.partial(
        pl.kernel,
        mesh=_mesh(),
        out_type=jax.ShapeDtypeStruct((_NC, npad), jnp.float32),
        scratch_types=[
            pltpu.VMEM((8, 128), jnp.int32),
            pltpu.VMEM((128,), jnp.float32),
            pltpu.VMEM_SHARED((npad,), jnp.float32),
        ],
    )
    def deg_kernel(e_hbm, zero_hbm, out_hbm, dstv, onesv, accum):
        cid = lax.axis_index("c")
        sid = lax.axis_index("s")
        wid = sid * _NC + cid
        pltpu.sync_copy(zero_hbm.at[pl.ds(sid * seg, seg)],
                        accum.at[pl.ds(sid * seg, seg)])
        one16 = jnp.full((_LANES,), 1.0, jnp.float32)
        for k in range(8):
            onesv[pl.ds(k * _LANES, _LANES)] = one16
        plsc.subcore_barrier()
        g0 = (wid * groups) // (_NC * _NS)
        g1 = ((wid + 1) * groups) // (_NC * _NS)

        @pl.loop(g0, g1)
        def _(g):
            pltpu.sync_copy(e_hbm.at[1, pl.ds(g * 8, 8)], dstv)
            for r in range(8):
                pltpu.sync_copy(onesv, accum.at[dstv.at[r]], add=True)

        plsc.subcore_barrier()
        pltpu.sync_copy(accum.at[pl.ds(sid * seg, seg)],
                        out_hbm.at[cid, pl.ds(sid * seg, seg)])

    return deg_kernel


def _make_agg_kernel(rows, npad):
    groups = rows // 8
    seg = npad // _NS

    @functools.partial(
        pl.kernel,
        mesh=_mesh(),
        out_type=jax.ShapeDtypeStruct((_NC, npad), jnp.float32),
        scratch_types=[
            pltpu.VMEM((8, 128), jnp.int32),
            pltpu.VMEM((8, 128), jnp.int32),
            pltpu.VMEM((128,), jnp.float32),
            pltpu.VMEM((npad,), jnp.float32),
            pltpu.VMEM_SHARED((npad,), jnp.float32),
        ],
        compiler_params=pltpu.CompilerParams(needs_layout_passes=False),
    )
    def agg_kernel(e_hbm, tab_hbm, zero_hbm, out_hbm, srcv, dstv, valsv, table,
                   accum):
        cid = lax.axis_index("c")
        sid = lax.axis_index("s")
        wid = sid * _NC + cid
        pltpu.sync_copy(zero_hbm.at[pl.ds(sid * seg, seg)],
                        accum.at[pl.ds(sid * seg, seg)])
        pltpu.sync_copy(tab_hbm, table)
        plsc.subcore_barrier()
        g0 = (wid * groups) // (_NC * _NS)
        g1 = ((wid + 1) * groups) // (_NC * _NS)

        @pl.loop(g0, g1)
        def _(g):
            pltpu.sync_copy(e_hbm.at[0, pl.ds(g * 8, 8)], srcv)
            pltpu.sync_copy(e_hbm.at[1, pl.ds(g * 8, 8)], dstv)
            for r in range(8):
                for c in range(8):
                    idx = srcv[r, pl.ds(c * _LANES, _LANES)]
                    vals = plsc.load_gather(table, [idx])
                    valsv[pl.ds(c * _LANES, _LANES)] = vals
                pltpu.sync_copy(valsv, accum.at[dstv.at[r]], add=True)

        plsc.subcore_barrier()
        pltpu.sync_copy(accum.at[pl.ds(sid * seg, seg)],
                        out_hbm.at[cid, pl.ds(sid * seg, seg)])

    return agg_kernel


def _tc_prep(deg_parts, xpad):
    r128 = xpad.shape

    def body(dref, xref, dis_ref, xd_ref):
        deg = dref[0] + dref[1]
        dis = jnp.where(deg > 0, lax.rsqrt(jnp.maximum(deg, 1e-12)),
                        jnp.zeros_like(deg))
        dis_ref[...] = dis
        xd_ref[...] = dis * xref[...]

    return pl.pallas_call(
        body,
        out_shape=(
            jax.ShapeDtypeStruct(r128, jnp.float32),
            jax.ShapeDtypeStruct(r128, jnp.float32),
        ),
    )(deg_parts, xpad)


def _tc_mid(t1_parts, dis, W1, b1, W2):
    r128 = dis.shape

    def body(tref, dis_ref, w1_ref, b1_ref, w2_ref, hd_ref):
        d = dis_ref[...]
        s1 = d * (tref[0] + tref[1])
        acc = jnp.zeros_like(s1)
        for j in range(16):
            acc = acc + jnp.maximum(s1 * w1_ref[0, j] + b1_ref[j], 0.0) * w2_ref[j, 0]
        hd_ref[...] = acc * d

    return pl.pallas_call(
        body,
        in_specs=[
            pl.BlockSpec(),
            pl.BlockSpec(),
            pl.BlockSpec(memory_space=pltpu.SMEM),
            pl.BlockSpec(memory_space=pltpu.SMEM),
            pl.BlockSpec(memory_space=pltpu.SMEM),
        ],
        out_shape=jax.ShapeDtypeStruct(r128, jnp.float32),
    )(t1_parts, dis, W1, b1, W2)


def _tc_final(t2_parts, dis, b2):
    r128 = dis.shape

    def body(tref, dis_ref, b2_ref, out_ref):
        out_ref[...] = dis_ref[...] * (tref[0] + tref[1]) + b2_ref[0]

    return pl.pallas_call(
        body,
        in_specs=[
            pl.BlockSpec(),
            pl.BlockSpec(),
            pl.BlockSpec(memory_space=pltpu.SMEM),
        ],
        out_shape=jax.ShapeDtypeStruct(r128, jnp.float32),
    )(t2_parts, dis, b2)


def kernel(x, edge_index, W1, b1, W2, b2):
    n = x.shape[0]
    e = edge_index.shape[1]
    assert e % 1024 == 0
    rows = e // 128
    npad = ((n + 1023) // 1024) * 1024
    r = npad // 128

    ei = edge_index.astype(jnp.int32).reshape(2, rows, 128)
    zeros_np = jnp.zeros((npad,), jnp.float32)
    xpad = jnp.concatenate([x[:, 0], jnp.zeros((npad - n,), jnp.float32)])

    deg_parts = _make_deg_kernel(rows, npad)(ei, zeros_np)
    dis, xd = _tc_prep(deg_parts.reshape(2, r, 128), xpad.reshape(r, 128))

    agg = _make_agg_kernel(rows, npad)
    t1_parts = agg(ei, xd.reshape(npad), zeros_np)
    hd = _tc_mid(t1_parts.reshape(2, r, 128), dis, W1, b1, W2)

    t2_parts = agg(ei, hd.reshape(npad), zeros_np)
    out = _tc_final(t2_parts.reshape(2, r, 128), dis, b2)

    return out.reshape(npad)[:n].reshape(n, 1)


# speedup vs baseline: 219.8534x; 1.1807x over previous
import functools

import jax
import jax.numpy as jnp
from jax import lax
from jax.experimental import pallas as pl
from jax.experimental.pallas import tpu as pltpu
from jax.experimental.pallas import tpu_sc as plsc

_NC = 2
_NS = 16
_LANES = 16


def _mesh():
    return plsc.VectorSubcoreMesh(
        core_axis_name="c", subcore_axis_name="s", num_cores=_NC, num_subcores=_NS
    )


def _make_deg_kernel(rows, npad):
    groups = rows
    seg = npad // _NS

    @functools.partial(
        pl.kernel,
        mesh=_mesh(),
        out_type=jax.ShapeDtypeStruct((_NC, npad), jnp.float32),
        scratch_types=[
            pltpu.VMEM((1024,), jnp.int32),
            pltpu.VMEM((1024,), jnp.float32),
            pltpu.VMEM_SHARED((npad,), jnp.float32),
        ],
    )
    def deg_kernel(e_hbm, zero_hbm, out_hbm, dstv, onesv, accum):
        cid = lax.axis_index("c")
        sid = lax.axis_index("s")
        wid = sid * _NC + cid
        pltpu.sync_copy(zero_hbm.at[pl.ds(sid * seg, seg)],
                        accum.at[pl.ds(sid * seg, seg)])
        one16 = jnp.full((_LANES,), 1.0, jnp.float32)
        for k in range(64):
            onesv[pl.ds(k * _LANES, _LANES)] = one16
        plsc.subcore_barrier()
        g0 = (wid * groups) // (_NC * _NS)
        g1 = ((wid + 1) * groups) // (_NC * _NS)

        @pl.loop(g0, g1)
        def _(g):
            pltpu.sync_copy(e_hbm.at[1, g], dstv)
            pltpu.sync_copy(onesv, accum.at[dstv], add=True)

        plsc.subcore_barrier()
        pltpu.sync_copy(accum.at[pl.ds(sid * seg, seg)],
                        out_hbm.at[cid, pl.ds(sid * seg, seg)])

    return deg_kernel


def _make_agg_kernel(rows, npad):
    groups = rows
    seg = npad // _NS

    @functools.partial(
        pl.kernel,
        mesh=_mesh(),
        out_type=jax.ShapeDtypeStruct((_NC, npad), jnp.float32),
        scratch_types=[
            pltpu.VMEM((1024,), jnp.int32),
            pltpu.VMEM((1024,), jnp.int32),
            pltpu.VMEM((1024,), jnp.float32),
            pltpu.VMEM((npad,), jnp.float32),
            pltpu.VMEM_SHARED((npad,), jnp.float32),
        ],
        compiler_params=pltpu.CompilerParams(needs_layout_passes=False),
    )
    def agg_kernel(e_hbm, tab_hbm, zero_hbm, out_hbm, srcv, dstv, valsv, table,
                   accum):
        cid = lax.axis_index("c")
        sid = lax.axis_index("s")
        wid = sid * _NC + cid
        pltpu.sync_copy(zero_hbm.at[pl.ds(sid * seg, seg)],
                        accum.at[pl.ds(sid * seg, seg)])
        pltpu.sync_copy(tab_hbm, table)
        plsc.subcore_barrier()
        g0 = (wid * groups) // (_NC * _NS)
        g1 = ((wid + 1) * groups) // (_NC * _NS)

        @pl.loop(g0, g1)
        def _(g):
            pltpu.sync_copy(e_hbm.at[0, g], srcv)
            pltpu.sync_copy(e_hbm.at[1, g], dstv)
            for c in range(64):
                idx = srcv[pl.ds(c * _LANES, _LANES)]
                vals = plsc.load_gather(table, [idx])
                valsv[pl.ds(c * _LANES, _LANES)] = vals
            pltpu.sync_copy(valsv, accum.at[dstv], add=True)

        plsc.subcore_barrier()
        pltpu.sync_copy(accum.at[pl.ds(sid * seg, seg)],
                        out_hbm.at[cid, pl.ds(sid * seg, seg)])

    return agg_kernel


def _tc_prep(deg_parts, xpad):
    r128 = xpad.shape

    def body(dref, xref, dis_ref, xd_ref):
        deg = dref[0] + dref[1]
        dis = jnp.where(deg > 0, lax.rsqrt(jnp.maximum(deg, 1e-12)),
                        jnp.zeros_like(deg))
        dis_ref[...] = dis
        xd_ref[...] = dis * xref[...]

    return pl.pallas_call(
        body,
        out_shape=(
            jax.ShapeDtypeStruct(r128, jnp.float32),
            jax.ShapeDtypeStruct(r128, jnp.float32),
        ),
    )(deg_parts, xpad)


def _tc_mid(t1_parts, dis, W1, b1, W2):
    r128 = dis.shape

    def body(tref, dis_ref, w1_ref, b1_ref, w2_ref, hd_ref):
        d = dis_ref[...]
        s1 = d * (tref[0] + tref[1])
        acc = jnp.zeros_like(s1)
        for j in range(16):
            acc = acc + jnp.maximum(s1 * w1_ref[0, j] + b1_ref[j], 0.0) * w2_ref[j, 0]
        hd_ref[...] = acc * d

    return pl.pallas_call(
        body,
        in_specs=[
            pl.BlockSpec(),
            pl.BlockSpec(),
            pl.BlockSpec(memory_space=pltpu.SMEM),
            pl.BlockSpec(memory_space=pltpu.SMEM),
            pl.BlockSpec(memory_space=pltpu.SMEM),
        ],
        out_shape=jax.ShapeDtypeStruct(r128, jnp.float32),
    )(t1_parts, dis, W1, b1, W2)


def _tc_final(t2_parts, dis, b2):
    r128 = dis.shape

    def body(tref, dis_ref, b2_ref, out_ref):
        out_ref[...] = dis_ref[...] * (tref[0] + tref[1]) + b2_ref[0]

    return pl.pallas_call(
        body,
        in_specs=[
            pl.BlockSpec(),
            pl.BlockSpec(),
            pl.BlockSpec(memory_space=pltpu.SMEM),
        ],
        out_shape=jax.ShapeDtypeStruct(r128, jnp.float32),
    )(t2_parts, dis, b2)


def kernel(x, edge_index, W1, b1, W2, b2):
    n = x.shape[0]
    e = edge_index.shape[1]
    assert e % 1024 == 0
    rows = e // 1024
    npad = ((n + 1023) // 1024) * 1024
    r = npad // 128

    ei = edge_index.astype(jnp.int32).reshape(2, rows, 1024)
    zeros_np = jnp.zeros((npad,), jnp.float32)
    xpad = jnp.concatenate([x[:, 0], jnp.zeros((npad - n,), jnp.float32)])

    deg_parts = _make_deg_kernel(rows, npad)(ei, zeros_np)
    dis, xd = _tc_prep(deg_parts.reshape(2, r, 128), xpad.reshape(r, 128))

    agg = _make_agg_kernel(rows, npad)
    t1_parts = agg(ei, xd.reshape(npad), zeros_np)
    hd = _tc_mid(t1_parts.reshape(2, r, 128), dis, W1, b1, W2)

    t2_parts = agg(ei, hd.reshape(npad), zeros_np)
    out = _tc_final(t2_parts.reshape(2, r, 128), dis, b2)

    return out.reshape(npad)[:n].reshape(n, 1)
